# Initial kernel scaffold; baseline (speedup 1.0000x reference)
import jax
import jax.numpy as jnp
from jax.experimental import pallas as pl


def kernel(atoms, distances, mu, sigma, a, b):
    raise NotImplementedError("write your pallas kernel here")


# speedup vs baseline: 41.5336x; 41.5336x over previous
import functools
from math import sqrt, pi

import jax
import jax.numpy as jnp
from jax import lax
from jax.experimental import pallas as pl
from jax.experimental.pallas import tpu as pltpu
from jax.experimental.pallas import tpu_sc as plsc

NKERNEL = 128
POSINF = 10.0
EPS = 1e-05

_B = 16
_N = 128
_ROWS = _B * _N
_NW = 32
_RPW = _ROWS // _NW
_TAB = 121 * 121
_TABPAD = 14656

_F32_MIN = jnp.finfo(jnp.float32).min


def _sc_gather_affine(atoms_flat, a_flat, b_flat, d2):
    mesh = plsc.VectorSubcoreMesh(core_axis_name="c", subcore_axis_name="s")

    @functools.partial(
        pl.kernel,
        out_type=jax.ShapeDtypeStruct((_ROWS, _N), jnp.float32),
        mesh=mesh,
        compiler_params=pltpu.CompilerParams(needs_layout_passes=False),
        scratch_types=[
            pltpu.VMEM((_TABPAD,), jnp.float32),
            pltpu.VMEM((_TABPAD,), jnp.float32),
            pltpu.VMEM((_N,), jnp.int32),
            pltpu.VMEM((_RPW, _N), jnp.float32),
            pltpu.VMEM((_RPW, _N), jnp.float32),
        ],
    )
    def k(atoms_hbm, a_hbm, b_hbm, d_hbm, x_hbm, a_v, b_v, at_v, d_v, x_v):
        wid = lax.axis_index("s") * 2 + lax.axis_index("c")
        row0 = wid * _RPW
        batch = wid // (_N // _RPW)
        i0 = (wid % (_N // _RPW)) * _RPW

        pltpu.sync_copy(a_hbm, a_v)
        pltpu.sync_copy(b_hbm, b_v)
        pltpu.sync_copy(atoms_hbm.at[pl.ds(batch * _N, _N)], at_v)
        pltpu.sync_copy(d_hbm.at[pl.ds(row0, _RPW)], d_v)

        def row_body(r, carry):
            i_splat = jnp.full((16,), i0, jnp.int32) + r
            base = plsc.load_gather(at_v, [i_splat]) * 121
            for jc in range(_N // 16):
                aj = at_v[pl.ds(jc * 16, 16)]
                idx = aj + base
                ag = plsc.load_gather(a_v, [idx])
                bg = plsc.load_gather(b_v, [idx])
                dv = d_v[r, pl.ds(jc * 16, 16)]
                dv = jnp.where(jnp.isnan(dv), jnp.float32(0.0), dv)
                dv = jnp.where(dv == jnp.inf, jnp.float32(POSINF), dv)
                dv = jnp.where(dv == -jnp.inf, _F32_MIN, dv)
                x_v[r, pl.ds(jc * 16, 16)] = ag * dv + bg
            return carry

        lax.fori_loop(0, _RPW, row_body, 0)
        pltpu.sync_copy(x_v, x_hbm.at[pl.ds(row0, _RPW)])

    return k(atoms_flat, a_flat, b_flat, d2)


def _tc_rbf(x2, mu2, sigma2):
    rows_per_block = 16
    grid = (_ROWS // rows_per_block,)

    def body(x_ref, mu_ref, sig_ref, o_ref):
        x = x_ref[...]
        mu = mu_ref[...].reshape(1, 1, NKERNEL)
        sig = sig_ref[...].reshape(1, 1, NKERNEL)
        inv = 1.0 / sig
        c = 1.0 / ((jnp.abs(sig) + EPS) * sqrt(2.0 * pi))
        t = (x[:, :, None] - mu) * inv
        o_ref[...] = jnp.exp(-0.5 * (t * t)) * c

    return pl.pallas_call(
        body,
        grid=grid,
        in_specs=[
            pl.BlockSpec((rows_per_block, _N), lambda i: (i, 0)),
            pl.BlockSpec((1, NKERNEL), lambda i: (0, 0)),
            pl.BlockSpec((1, NKERNEL), lambda i: (0, 0)),
        ],
        out_specs=pl.BlockSpec((rows_per_block, _N, NKERNEL),
                               lambda i: (i, 0, 0)),
        out_shape=jax.ShapeDtypeStruct((_ROWS, _N, NKERNEL), jnp.float32),
    )(x2, mu2, sigma2)


@jax.jit
def kernel(atoms, distances, mu, sigma, a, b):
    atoms_flat = atoms.reshape(-1).astype(jnp.int32)
    a_flat = jnp.pad(a.reshape(-1), (0, _TABPAD - _TAB))
    b_flat = jnp.pad(b.reshape(-1), (0, _TABPAD - _TAB))
    d2 = distances.reshape(_ROWS, _N)
    x2 = _sc_gather_affine(atoms_flat, a_flat, b_flat, d2)
    g = _tc_rbf(x2, mu.reshape(1, NKERNEL), sigma.reshape(1, NKERNEL))
    return g.reshape(_B, _N, _N, NKERNEL)


# speedup vs baseline: 54.1562x; 1.3039x over previous
import functools
from math import sqrt, pi

import jax
import jax.numpy as jnp
from jax import lax
from jax.experimental import pallas as pl
from jax.experimental.pallas import tpu as pltpu
from jax.experimental.pallas import tpu_sc as plsc

NKERNEL = 128
POSINF = 10.0
EPS = 1e-05

_B = 16
_N = 128
_ROWS = _B * _N
_NW = 32
_RPW = _ROWS // _NW
_TAB = 121 * 121
_TABPAD = 14656

_F32_MIN = jnp.finfo(jnp.float32).min


def _sc_gather_affine(atoms_flat, a_flat, b_flat, d2):
    mesh = plsc.VectorSubcoreMesh(core_axis_name="c", subcore_axis_name="s")

    @functools.partial(
        pl.kernel,
        out_type=jax.ShapeDtypeStruct((_ROWS, _N), jnp.float32),
        mesh=mesh,
        compiler_params=pltpu.CompilerParams(needs_layout_passes=False),
        scratch_types=[
            pltpu.VMEM((_TABPAD,), jnp.float32),
            pltpu.VMEM((_TABPAD,), jnp.float32),
            pltpu.VMEM((_N,), jnp.int32),
            pltpu.VMEM((_RPW, _N), jnp.float32),
            pltpu.VMEM((_RPW, _N), jnp.float32),
        ],
    )
    def k(atoms_hbm, a_hbm, b_hbm, d_hbm, x_hbm, a_v, b_v, at_v, d_v, x_v):
        wid = lax.axis_index("s") * 2 + lax.axis_index("c")
        row0 = wid * _RPW
        batch = wid // (_N // _RPW)
        i0 = (wid % (_N // _RPW)) * _RPW

        pltpu.sync_copy(a_hbm, a_v)
        pltpu.sync_copy(b_hbm, b_v)
        pltpu.sync_copy(atoms_hbm.at[pl.ds(batch * _N, _N)], at_v)
        pltpu.sync_copy(d_hbm.at[pl.ds(row0, _RPW)], d_v)

        def row_body(r, carry):
            i_splat = jnp.full((16,), i0, jnp.int32) + r
            base = plsc.load_gather(at_v, [i_splat]) * 121
            for jc in range(_N // 16):
                aj = at_v[pl.ds(jc * 16, 16)]
                idx = aj + base
                ag = plsc.load_gather(a_v, [idx])
                bg = plsc.load_gather(b_v, [idx])
                dv = d_v[r, pl.ds(jc * 16, 16)]
                dv = jnp.where(jnp.isnan(dv), jnp.float32(0.0), dv)
                dv = jnp.where(dv == jnp.inf, jnp.float32(POSINF), dv)
                dv = jnp.where(dv == -jnp.inf, _F32_MIN, dv)
                x_v[r, pl.ds(jc * 16, 16)] = ag * dv + bg
            return carry

        lax.fori_loop(0, _RPW, row_body, 0)
        pltpu.sync_copy(x_v, x_hbm.at[pl.ds(row0, _RPW)])

    return k(atoms_flat, a_flat, b_flat, d2)


_LOG2E = 1.4426950408889634


def _tc_rbf(x2, mu2, sigma2):
    rows_per_block = 32
    grid = (_ROWS // rows_per_block,)

    def body(x_ref, mu_ref, sig_ref, o_ref, const_ref):
        @pl.when(pl.program_id(0) == 0)
        def _():
            sig = sig_ref[...]
            const_ref[0:1, :] = mu_ref[...]
            const_ref[1:2, :] = (-0.5 * _LOG2E) / (sig * sig)
            const_ref[2:3, :] = -jnp.log2((jnp.abs(sig) + EPS) * sqrt(2.0 * pi))

        mu = const_ref[0:1, :].reshape(1, 1, NKERNEL)
        s2 = const_ref[1:2, :].reshape(1, 1, NKERNEL)
        lc = const_ref[2:3, :].reshape(1, 1, NKERNEL)
        x = x_ref[...]
        u = x[:, :, None] - mu
        o_ref[...] = jnp.exp2((u * u) * s2 + lc)

    return pl.pallas_call(
        body,
        grid=grid,
        in_specs=[
            pl.BlockSpec((rows_per_block, _N), lambda i: (i, 0)),
            pl.BlockSpec((1, NKERNEL), lambda i: (0, 0)),
            pl.BlockSpec((1, NKERNEL), lambda i: (0, 0)),
        ],
        out_specs=pl.BlockSpec((rows_per_block, _N, NKERNEL),
                               lambda i: (i, 0, 0)),
        out_shape=jax.ShapeDtypeStruct((_ROWS, _N, NKERNEL), jnp.float32),
        scratch_shapes=[pltpu.VMEM((8, NKERNEL), jnp.float32)],
    )(x2, mu2, sigma2)


@jax.jit
def kernel(atoms, distances, mu, sigma, a, b):
    atoms_flat = atoms.reshape(-1).astype(jnp.int32)
    a_flat = jnp.pad(a.reshape(-1), (0, _TABPAD - _TAB))
    b_flat = jnp.pad(b.reshape(-1), (0, _TABPAD - _TAB))
    d2 = distances.reshape(_ROWS, _N)
    x2 = _sc_gather_affine(atoms_flat, a_flat, b_flat, d2)
    g = _tc_rbf(x2, mu.reshape(1, NKERNEL), sigma.reshape(1, NKERNEL))
    return g.reshape(_B, _N, _N, NKERNEL)


# speedup vs baseline: 57.2754x; 1.0576x over previous
import functools
from math import sqrt, pi

import jax
import jax.numpy as jnp
from jax import lax
from jax.experimental import pallas as pl
from jax.experimental.pallas import tpu as pltpu
from jax.experimental.pallas import tpu_sc as plsc

NKERNEL = 128
POSINF = 10.0
EPS = 1e-05

_B = 16
_N = 128
_ROWS = _B * _N
_NW = 32
_RPW = _ROWS // _NW
_NA = 121
_NAPAD = 128

_F32_MIN = jnp.finfo(jnp.float32).min


def _sc_gather_affine(atoms_flat, a_pad, b_pad, d2):
    mesh = plsc.VectorSubcoreMesh(core_axis_name="c", subcore_axis_name="s")

    @functools.partial(
        pl.kernel,
        out_type=jax.ShapeDtypeStruct((_ROWS, _N), jnp.float32),
        mesh=mesh,
        compiler_params=pltpu.CompilerParams(needs_layout_passes=False),
        scratch_types=[
            pltpu.VMEM((_NA, _NAPAD), jnp.float32),
            pltpu.VMEM((_NA, _NAPAD), jnp.float32),
            pltpu.VMEM((_N,), jnp.int32),
            pltpu.VMEM((_RPW, _N), jnp.float32),
            pltpu.VMEM((_RPW, _N), jnp.float32),
        ],
    )
    def k(atoms_hbm, a_hbm, b_hbm, d_hbm, x_hbm, a_v, b_v, at_v, d_v, x_v):
        wid = lax.axis_index("s") * 2 + lax.axis_index("c")
        row0 = wid * _RPW
        batch = wid // (_N // _RPW)
        i0 = (wid % (_N // _RPW)) * _RPW

        pltpu.sync_copy(a_hbm, a_v)
        pltpu.sync_copy(b_hbm, b_v)
        pltpu.sync_copy(atoms_hbm.at[pl.ds(batch * _N, _N)], at_v)
        pltpu.sync_copy(d_hbm.at[pl.ds(row0, _RPW)], d_v)

        ajs = [at_v[pl.ds(jc * 16, 16)] for jc in range(_N // 16)]

        def row_body(r):
            i_splat = jnp.full((16,), i0, jnp.int32) + r
            ai = plsc.load_gather(at_v, [i_splat])
            for jc, aj in enumerate(ajs):
                ag = plsc.load_gather(a_v, [ai, aj])
                bg = plsc.load_gather(b_v, [ai, aj])
                dv = d_v[r, pl.ds(jc * 16, 16)]
                dv = jnp.where(jnp.isnan(dv), jnp.float32(0.0), dv)
                dv = jnp.where(dv == jnp.inf, jnp.float32(POSINF), dv)
                dv = jnp.where(dv == -jnp.inf, _F32_MIN, dv)
                x_v[r, pl.ds(jc * 16, 16)] = ag * dv + bg

        plsc.parallel_loop(0, _RPW, 1, unroll=2)(row_body)
        pltpu.sync_copy(x_v, x_hbm.at[pl.ds(row0, _RPW)])

    return k(atoms_flat, a_pad, b_pad, d2)


_LOG2E = 1.4426950408889634


def _tc_rbf(x2, mu2, sigma2):
    rows_per_block = 32
    grid = (_ROWS // rows_per_block,)

    def body(x_ref, mu_ref, sig_ref, o_ref, const_ref):
        @pl.when(pl.program_id(0) == 0)
        def _():
            sig = sig_ref[...]
            const_ref[0:1, :] = mu_ref[...]
            const_ref[1:2, :] = (-0.5 * _LOG2E) / (sig * sig)
            const_ref[2:3, :] = -jnp.log2((jnp.abs(sig) + EPS) * sqrt(2.0 * pi))

        mu = const_ref[0:1, :].reshape(1, 1, NKERNEL)
        s2 = const_ref[1:2, :].reshape(1, 1, NKERNEL)
        lc = const_ref[2:3, :].reshape(1, 1, NKERNEL)
        x = x_ref[...]
        u = x[:, :, None] - mu
        o_ref[...] = jnp.exp2((u * u) * s2 + lc)

    return pl.pallas_call(
        body,
        grid=grid,
        in_specs=[
            pl.BlockSpec((rows_per_block, _N), lambda i: (i, 0)),
            pl.BlockSpec((1, NKERNEL), lambda i: (0, 0)),
            pl.BlockSpec((1, NKERNEL), lambda i: (0, 0)),
        ],
        out_specs=pl.BlockSpec((rows_per_block, _N, NKERNEL),
                               lambda i: (i, 0, 0)),
        out_shape=jax.ShapeDtypeStruct((_ROWS, _N, NKERNEL), jnp.float32),
        scratch_shapes=[pltpu.VMEM((8, NKERNEL), jnp.float32)],
    )(x2, mu2, sigma2)


@jax.jit
def kernel(atoms, distances, mu, sigma, a, b):
    atoms_flat = atoms.reshape(-1).astype(jnp.int32)
    a_pad = jnp.pad(a, ((0, 0), (0, _NAPAD - _NA)))
    b_pad = jnp.pad(b, ((0, 0), (0, _NAPAD - _NA)))
    d2 = distances.reshape(_ROWS, _N)
    x2 = _sc_gather_affine(atoms_flat, a_pad, b_pad, d2)
    g = _tc_rbf(x2, mu.reshape(1, NKERNEL), sigma.reshape(1, NKERNEL))
    return g.reshape(_B, _N, _N, NKERNEL)
